# baseline (speedup 1.0000x reference)
import functools

import jax
import jax.numpy as jnp
import numpy as np
from jax import lax
from jax.experimental import pallas as pl
from jax.experimental.pallas import tpu as pltpu
from jax.experimental.pallas import tpu_sc as plsc

_N = 10000
_E = 160000
_DE = 384
_H = 128
_NCLS = 64
_BE = 640
_BN = 1000
_GSZ = 64
_NGE = _E // _GSZ
_GMAX = 80
_NGP = _GMAX * 32
_NA = 10112


@functools.lru_cache(maxsize=None)
def _sc_message_pass(h):
    mesh = plsc.VectorSubcoreMesh(core_axis_name="c", subcore_axis_name="s")

    @functools.partial(
        pl.kernel,
        out_type=jax.ShapeDtypeStruct((2, _N, h), jnp.float32),
        mesh=mesh,
        scratch_types=[
            pltpu.VMEM((_GMAX * _GSZ,), jnp.int32),
            pltpu.VMEM((_GMAX, _GSZ), jnp.int32),
            pltpu.VMEM((_GSZ, h), jnp.float32),
            pltpu.VMEM((_GSZ, h), jnp.float32),
            pltpu.VMEM((_GSZ, h), jnp.float32),
            pltpu.VMEM((_GSZ, h), jnp.float32),
            pltpu.VMEM_SHARED((_NA, h), jnp.float32),
            pltpu.SemaphoreType.DMA,
            pltpu.SemaphoreType.DMA,
            pltpu.SemaphoreType.DMA,
            pltpu.SemaphoreType.DMA,
        ],
    )
    def mp(xt_hbm, ea_hbm, row_hbm, col_hbm, out_hbm,
           row_v, col_v, ea0, ea1, xr0, xr1, acc,
           gs0, gs1, es0, es1):
        eav = (ea0, ea1)
        xrv = (xr0, xr1)
        gsem = (gs0, gs1)
        esem = (es0, es1)
        c = lax.axis_index("c")
        s = lax.axis_index("s")
        w = c * 16 + s

        def zrow(r, carry):
            for u in range(h // 16):
                xr0[r, pl.ds(u * 16, 16)] = jnp.zeros((16,), jnp.float32)
            return carry
        lax.fori_loop(0, _GSZ, zrow, 0)
        zc_base = s * 10
        zc_cnt = jnp.minimum(10, 158 - zc_base)

        def zchunk(i, carry):
            pltpu.sync_copy(xr0, acc.at[pl.ds((zc_base + i) * _GSZ, _GSZ)])
            return carry
        lax.fori_loop(0, zc_cnt, zchunk, 0)
        plsc.subcore_barrier()

        pltpu.sync_copy(row_hbm.at[pl.ds(w * _GMAX * _GSZ, _GMAX * _GSZ)],
                        row_v)
        pltpu.sync_copy(col_hbm.at[pl.ds(w * _GMAX, _GMAX)], col_v)

        def gather_copy(k, j):
            return pltpu.make_async_copy(
                xt_hbm.at[row_v.at[pl.ds(k * _GSZ, _GSZ)]], xrv[j], gsem[j])

        def ea_copy(k, j):
            base = jnp.minimum((w * _GMAX + k) * _GSZ, _E - _GSZ)
            return pltpu.make_async_copy(
                ea_hbm.at[pl.ds(base, _GSZ)], eav[j], esem[j])

        gather_copy(0, 0).start()
        ea_copy(0, 0).start()
        gather_copy(1, 1).start()
        ea_copy(1, 1).start()

        def body(i, carry):
            for j in (0, 1):
                k = 2 * i + j
                gather_copy(k, j).wait()
                ea_copy(k, j).wait()

                @plsc.parallel_loop(0, _GSZ, 1, unroll=4)
                def mul(r):
                    for u in range(h // 16):
                        sl = pl.ds(16 * u, 16)
                        xrv[j][r, sl] = eav[j][r, sl] * xrv[j][r, sl]

                @pl.when(i < _GMAX // 2 - 1)
                def _():
                    ea_copy(k + 2, j).start()
                pltpu.sync_copy(xrv[j], acc.at[col_v.at[k]], add=True)

                @pl.when(i < _GMAX // 2 - 1)
                def _():
                    gather_copy(k + 2, j).start()
            return carry
        lax.fori_loop(0, _GMAX // 2, body, 0)

        plsc.subcore_barrier()
        oc_base = s * 8
        oc_cnt = jnp.minimum(8, 125 - oc_base)

        def ochunk(i, carry):
            pltpu.sync_copy(acc.at[pl.ds((oc_base + i) * 80, 80)],
                            out_hbm.at[c, pl.ds((oc_base + i) * 80, 80)])
            return carry
        lax.fori_loop(0, oc_cnt, ochunk, 0)

    return mp


def _silu(a):
    return a / (1.0 + jnp.exp(-a))


def _spline_bases(a):
    v = 2.0 * a + 5.0
    c = jnp.floor(v)
    t = v - c
    t2 = t * t
    t3 = t2 * t
    w0 = t3 * (1.0 / 6.0)
    w1 = (1.0 + 3.0 * t + 3.0 * t2 - 3.0 * t3) * (1.0 / 6.0)
    w2 = (4.0 - 6.0 * t2 + 3.0 * t3) * (1.0 / 6.0)
    om = 1.0 - t
    w3 = om * om * om * (1.0 / 6.0)
    zero = jnp.zeros_like(a)
    out = []
    for j in range(7):
        b = (jnp.where(c == j, w0, zero) + jnp.where(c == j + 1, w1, zero)
             + jnp.where(c == j + 2, w2, zero) + jnp.where(c == j + 3, w3, zero))
        out.append(b)
    return out


def _kan(a, bwT_ref, swT_ref):
    out = jnp.dot(_silu(a), bwT_ref[...], preferred_element_type=jnp.float32)
    for j, b in enumerate(_spline_bases(a)):
        out = out + jnp.dot(b, swT_ref[j], preferred_element_type=jnp.float32)
    return out


def _eproj_body(ea_ref, w1, w2, w3, b1, b2, b3, o1, o2, o3):
    a = ea_ref[...]
    o1[...] = jnp.dot(a, w1[...], preferred_element_type=jnp.float32) + b1[...]
    o2[...] = jnp.dot(a, w2[...], preferred_element_type=jnp.float32) + b2[...]
    o3[...] = jnp.dot(a, w3[...], preferred_element_type=jnp.float32) + b3[...]


_eproj = pl.pallas_call(
    _eproj_body,
    grid=(_E // _BE,),
    in_specs=[
        pl.BlockSpec((_BE, _DE), lambda i: (i, 0)),
        pl.BlockSpec((_DE, _H), lambda i: (0, 0)),
        pl.BlockSpec((_DE, _H), lambda i: (0, 0)),
        pl.BlockSpec((_DE, _H), lambda i: (0, 0)),
        pl.BlockSpec((1, _H), lambda i: (0, 0)),
        pl.BlockSpec((1, _H), lambda i: (0, 0)),
        pl.BlockSpec((1, _H), lambda i: (0, 0)),
    ],
    out_specs=[
        pl.BlockSpec((_BE, _H), lambda i: (i, 0)),
        pl.BlockSpec((_BE, _H), lambda i: (i, 0)),
        pl.BlockSpec((_BE, _H), lambda i: (i, 0)),
    ],
    out_shape=[
        jax.ShapeDtypeStruct((_E, _H), jnp.float32),
        jax.ShapeDtypeStruct((_E, _H), jnp.float32),
        jax.ShapeDtypeStruct((_E, _H), jnp.float32),
    ],
)


def _nproj_body(x_ref, w_ref, b_ref, o_ref):
    o_ref[...] = (jnp.dot(x_ref[...], w_ref[...],
                          preferred_element_type=jnp.float32) + b_ref[...])


_nproj = pl.pallas_call(
    _nproj_body,
    grid=(_N // _BN,),
    in_specs=[
        pl.BlockSpec((_BN, _H), lambda i: (i, 0)),
        pl.BlockSpec((_H, _H), lambda i: (0, 0)),
        pl.BlockSpec((1, _H), lambda i: (0, 0)),
    ],
    out_specs=pl.BlockSpec((_BN, _H), lambda i: (i, 0)),
    out_shape=jax.ShapeDtypeStruct((_N, _H), jnp.float32),
)


def _kanb1_body(p_ref, bwT, swT, nW, nb, x1_ref, xt2_ref):
    a = p_ref[0] + p_ref[1]
    o = _kan(a, bwT, swT)
    x1_ref[...] = o
    xt2_ref[...] = (jnp.dot(o, nW[...], preferred_element_type=jnp.float32)
                    + nb[...])


_kanb1 = pl.pallas_call(
    _kanb1_body,
    grid=(_N // _BN,),
    in_specs=[
        pl.BlockSpec((2, _BN, _H), lambda i: (0, i, 0)),
        pl.BlockSpec((_H, _H), lambda i: (0, 0)),
        pl.BlockSpec((7, _H, _H), lambda i: (0, 0, 0)),
        pl.BlockSpec((_H, _H), lambda i: (0, 0)),
        pl.BlockSpec((1, _H), lambda i: (0, 0)),
    ],
    out_specs=[
        pl.BlockSpec((_BN, _H), lambda i: (i, 0)),
        pl.BlockSpec((_BN, _H), lambda i: (i, 0)),
    ],
    out_shape=[
        jax.ShapeDtypeStruct((_N, _H), jnp.float32),
        jax.ShapeDtypeStruct((_N, _H), jnp.float32),
    ],
)


def _kanb2_body(p_ref, bwT, swT, x_ref, x1_ref, wa, wb, wc, ob, xt3_ref):
    a = p_ref[0] + p_ref[1]
    x2 = _kan(a, bwT, swT)
    xt3_ref[...] = (jnp.dot(x_ref[...], wa[...], preferred_element_type=jnp.float32)
                    + jnp.dot(x1_ref[...], wb[...], preferred_element_type=jnp.float32)
                    + jnp.dot(x2, wc[...], preferred_element_type=jnp.float32)
                    + ob[...])


_kanb2 = pl.pallas_call(
    _kanb2_body,
    grid=(_N // _BN,),
    in_specs=[
        pl.BlockSpec((2, _BN, _H), lambda i: (0, i, 0)),
        pl.BlockSpec((_H, _H), lambda i: (0, 0)),
        pl.BlockSpec((7, _H, _H), lambda i: (0, 0, 0)),
        pl.BlockSpec((_BN, _H), lambda i: (i, 0)),
        pl.BlockSpec((_BN, _H), lambda i: (i, 0)),
        pl.BlockSpec((_H, _H), lambda i: (0, 0)),
        pl.BlockSpec((_H, _H), lambda i: (0, 0)),
        pl.BlockSpec((_H, _H), lambda i: (0, 0)),
        pl.BlockSpec((1, _H), lambda i: (0, 0)),
    ],
    out_specs=pl.BlockSpec((_BN, _H), lambda i: (i, 0)),
    out_shape=jax.ShapeDtypeStruct((_N, _H), jnp.float32),
)


def _kanc_body(p_ref, bwT, swT, out_ref):
    a = (p_ref[0] + p_ref[1])[:, :_NCLS]
    out_ref[...] = _kan(a, bwT, swT)


_kanc = pl.pallas_call(
    _kanc_body,
    grid=(_N // _BN,),
    in_specs=[
        pl.BlockSpec((2, _BN, _H), lambda i: (0, i, 0)),
        pl.BlockSpec((_NCLS, _NCLS), lambda i: (0, 0)),
        pl.BlockSpec((7, _NCLS, _NCLS), lambda i: (0, 0, 0)),
    ],
    out_specs=pl.BlockSpec((_BN, _NCLS), lambda i: (i, 0)),
    out_shape=jax.ShapeDtypeStruct((_N, _NCLS), jnp.float32),
)


def kernel(x, edge_index, edge_attr,
           l1_eW, l1_eb, l1_nW, l1_nb, l1_bw, l1_sw,
           l2_eW, l2_eb, l2_nW, l2_nb, l2_bw, l2_sw,
           o_eW, o_eb, o_nW, o_nb, o_bw, o_sw,
           grid_h, grid_out):
    npad = _NGP * _GSZ - _E
    ar = jnp.arange(npad, dtype=jnp.int32)
    row1d = jnp.concatenate([edge_index[0], ar % _N])
    col1d = jnp.concatenate([edge_index[1], _N + ar % 112]).reshape(_NGP, _GSZ)

    pad_w = ((0, 0), (0, _H - _NCLS))
    o_eW_p = jnp.pad(o_eW, pad_w)
    o_eb_p = jnp.pad(o_eb, (0, _H - _NCLS))
    o_nW_p = jnp.pad(o_nW, pad_w)
    o_nb_p = jnp.pad(o_nb, (0, _H - _NCLS))

    ea1, ea2, ea3 = _eproj(edge_attr, l1_eW, l2_eW, o_eW_p,
                           l1_eb.reshape(1, _H), l2_eb.reshape(1, _H),
                           o_eb_p.reshape(1, _H))

    _sc128 = _sc_message_pass(_H)

    xt1 = _nproj(x, l1_nW, l1_nb.reshape(1, _H))
    p1 = _sc128(xt1, ea1, row1d, col1d)
    x1, xt2 = _kanb1(p1, l1_bw.T, jnp.transpose(l1_sw, (2, 1, 0)),
                     l2_nW, l2_nb.reshape(1, _H))
    p2 = _sc128(xt2, ea2, row1d, col1d)
    xt3 = _kanb2(p2, l2_bw.T, jnp.transpose(l2_sw, (2, 1, 0)), x, x1,
                 o_nW_p[:_H], o_nW_p[_H:2 * _H], o_nW_p[2 * _H:],
                 o_nb_p.reshape(1, _H))
    p3 = _sc128(xt3, ea3, row1d, col1d)
    return _kanc(p3, o_bw.T, jnp.transpose(o_sw, (2, 1, 0)))


# speedup vs baseline: 1.1614x; 1.1614x over previous
import functools

import jax
import jax.numpy as jnp
import numpy as np
from jax import lax
from jax.experimental import pallas as pl
from jax.experimental.pallas import tpu as pltpu
from jax.experimental.pallas import tpu_sc as plsc

_N = 10000
_E = 160000
_DE = 384
_H = 128
_NCLS = 64
_BE = 1600
_BN = 1000
_GSZ = 64
_NGE = _E // _GSZ
_GMAX = 80
_NGP = _GMAX * 32
_NA = 10112


@functools.lru_cache(maxsize=None)
def _sc_message_pass(h):
    mesh = plsc.VectorSubcoreMesh(core_axis_name="c", subcore_axis_name="s")

    @functools.partial(
        pl.kernel,
        out_type=jax.ShapeDtypeStruct((2, _N, h), jnp.float32),
        mesh=mesh,
        scratch_types=[
            pltpu.VMEM((_GMAX * _GSZ,), jnp.int32),
            pltpu.VMEM((_GMAX, _GSZ), jnp.int32),
            pltpu.VMEM((_GSZ, h), jnp.float32),
            pltpu.VMEM((_GSZ, h), jnp.float32),
            pltpu.VMEM((_GSZ, h), jnp.float32),
            pltpu.VMEM((_GSZ, h), jnp.float32),
            pltpu.VMEM_SHARED((_NA, h), jnp.float32),
            pltpu.SemaphoreType.DMA,
            pltpu.SemaphoreType.DMA,
            pltpu.SemaphoreType.DMA,
            pltpu.SemaphoreType.DMA,
        ],
    )
    def mp(xt_hbm, ea_hbm, row_hbm, col_hbm, out_hbm,
           row_v, col_v, ea0, ea1, xr0, xr1, acc,
           gs0, gs1, es0, es1):
        eav = (ea0, ea1)
        xrv = (xr0, xr1)
        gsem = (gs0, gs1)
        esem = (es0, es1)
        c = lax.axis_index("c")
        s = lax.axis_index("s")
        w = c * 16 + s

        def zrow(r, carry):
            for u in range(h // 16):
                xr0[r, pl.ds(u * 16, 16)] = jnp.zeros((16,), jnp.float32)
            return carry
        lax.fori_loop(0, _GSZ, zrow, 0)
        zc_base = s * 10
        zc_cnt = jnp.minimum(10, 158 - zc_base)

        def zchunk(i, carry):
            pltpu.sync_copy(xr0, acc.at[pl.ds((zc_base + i) * _GSZ, _GSZ)])
            return carry
        lax.fori_loop(0, zc_cnt, zchunk, 0)
        plsc.subcore_barrier()

        pltpu.sync_copy(row_hbm.at[pl.ds(w * _GMAX * _GSZ, _GMAX * _GSZ)],
                        row_v)
        pltpu.sync_copy(col_hbm.at[pl.ds(w * _GMAX, _GMAX)], col_v)

        def gather_copy(k, j):
            return pltpu.make_async_copy(
                xt_hbm.at[row_v.at[pl.ds(k * _GSZ, _GSZ)]], xrv[j], gsem[j])

        def ea_copy(k, j):
            base = jnp.minimum((w * _GMAX + k) * _GSZ, _E - _GSZ)
            return pltpu.make_async_copy(
                ea_hbm.at[pl.ds(base, _GSZ)], eav[j], esem[j])

        gather_copy(0, 0).start()
        ea_copy(0, 0).start()
        gather_copy(1, 1).start()
        ea_copy(1, 1).start()

        def body(i, carry):
            for j in (0, 1):
                k = 2 * i + j
                gather_copy(k, j).wait()
                ea_copy(k, j).wait()

                def mul(r, carry2):
                    for u in range(h // 16):
                        sl = pl.ds(16 * u, 16)
                        xrv[j][r, sl] = eav[j][r, sl] * xrv[j][r, sl]
                    return carry2
                lax.fori_loop(0, _GSZ, mul, 0)

                @pl.when(i < _GMAX // 2 - 1)
                def _():
                    ea_copy(k + 2, j).start()
                pltpu.sync_copy(xrv[j], acc.at[col_v.at[k]], add=True)

                @pl.when(i < _GMAX // 2 - 1)
                def _():
                    gather_copy(k + 2, j).start()
            return carry
        lax.fori_loop(0, _GMAX // 2, body, 0)

        plsc.subcore_barrier()
        oc_base = s * 8
        oc_cnt = jnp.minimum(8, 125 - oc_base)

        def ochunk(i, carry):
            pltpu.sync_copy(acc.at[pl.ds((oc_base + i) * 80, 80)],
                            out_hbm.at[c, pl.ds((oc_base + i) * 80, 80)])
            return carry
        lax.fori_loop(0, oc_cnt, ochunk, 0)

    return mp


def _silu(a):
    return a / (1.0 + jnp.exp(-a))


def _spline_bases(a):
    v = 2.0 * a + 5.0
    c = jnp.floor(v)
    t = v - c
    t2 = t * t
    t3 = t2 * t
    w0 = t3 * (1.0 / 6.0)
    w1 = (1.0 + 3.0 * t + 3.0 * t2 - 3.0 * t3) * (1.0 / 6.0)
    w2 = (4.0 - 6.0 * t2 + 3.0 * t3) * (1.0 / 6.0)
    om = 1.0 - t
    w3 = om * om * om * (1.0 / 6.0)
    zero = jnp.zeros_like(a)
    out = []
    for j in range(7):
        b = (jnp.where(c == j, w0, zero) + jnp.where(c == j + 1, w1, zero)
             + jnp.where(c == j + 2, w2, zero) + jnp.where(c == j + 3, w3, zero))
        out.append(b)
    return out


def _kan(a, bwT_ref, swT_ref):
    out = jnp.dot(_silu(a), bwT_ref[...], preferred_element_type=jnp.float32)
    for j, b in enumerate(_spline_bases(a)):
        out = out + jnp.dot(b, swT_ref[j], preferred_element_type=jnp.float32)
    return out


def _eproj_body(ea_ref, w1, w2, w3, b1, b2, b3, o1, o2, o3):
    a = ea_ref[...]
    o1[...] = jnp.dot(a, w1[...], preferred_element_type=jnp.float32) + b1[...]
    o2[...] = jnp.dot(a, w2[...], preferred_element_type=jnp.float32) + b2[...]
    o3[...] = jnp.dot(a, w3[...], preferred_element_type=jnp.float32) + b3[...]


_eproj = pl.pallas_call(
    _eproj_body,
    grid=(_E // _BE,),
    in_specs=[
        pl.BlockSpec((_BE, _DE), lambda i: (i, 0)),
        pl.BlockSpec((_DE, _H), lambda i: (0, 0)),
        pl.BlockSpec((_DE, _H), lambda i: (0, 0)),
        pl.BlockSpec((_DE, _H), lambda i: (0, 0)),
        pl.BlockSpec((1, _H), lambda i: (0, 0)),
        pl.BlockSpec((1, _H), lambda i: (0, 0)),
        pl.BlockSpec((1, _H), lambda i: (0, 0)),
    ],
    out_specs=[
        pl.BlockSpec((_BE, _H), lambda i: (i, 0)),
        pl.BlockSpec((_BE, _H), lambda i: (i, 0)),
        pl.BlockSpec((_BE, _H), lambda i: (i, 0)),
    ],
    out_shape=[
        jax.ShapeDtypeStruct((_E, _H), jnp.float32),
        jax.ShapeDtypeStruct((_E, _H), jnp.float32),
        jax.ShapeDtypeStruct((_E, _H), jnp.float32),
    ],
)


def _nproj_body(x_ref, w_ref, b_ref, o_ref):
    o_ref[...] = (jnp.dot(x_ref[...], w_ref[...],
                          preferred_element_type=jnp.float32) + b_ref[...])


_nproj = pl.pallas_call(
    _nproj_body,
    grid=(_N // _BN,),
    in_specs=[
        pl.BlockSpec((_BN, _H), lambda i: (i, 0)),
        pl.BlockSpec((_H, _H), lambda i: (0, 0)),
        pl.BlockSpec((1, _H), lambda i: (0, 0)),
    ],
    out_specs=pl.BlockSpec((_BN, _H), lambda i: (i, 0)),
    out_shape=jax.ShapeDtypeStruct((_N, _H), jnp.float32),
)


def _kanb1_body(p_ref, bwT, swT, nW, nb, x1_ref, xt2_ref):
    a = p_ref[0] + p_ref[1]
    o = _kan(a, bwT, swT)
    x1_ref[...] = o
    xt2_ref[...] = (jnp.dot(o, nW[...], preferred_element_type=jnp.float32)
                    + nb[...])


_kanb1 = pl.pallas_call(
    _kanb1_body,
    grid=(_N // _BN,),
    in_specs=[
        pl.BlockSpec((2, _BN, _H), lambda i: (0, i, 0)),
        pl.BlockSpec((_H, _H), lambda i: (0, 0)),
        pl.BlockSpec((7, _H, _H), lambda i: (0, 0, 0)),
        pl.BlockSpec((_H, _H), lambda i: (0, 0)),
        pl.BlockSpec((1, _H), lambda i: (0, 0)),
    ],
    out_specs=[
        pl.BlockSpec((_BN, _H), lambda i: (i, 0)),
        pl.BlockSpec((_BN, _H), lambda i: (i, 0)),
    ],
    out_shape=[
        jax.ShapeDtypeStruct((_N, _H), jnp.float32),
        jax.ShapeDtypeStruct((_N, _H), jnp.float32),
    ],
)


def _kanb2_body(p_ref, bwT, swT, x_ref, x1_ref, wa, wb, wc, ob, xt3_ref):
    a = p_ref[0] + p_ref[1]
    x2 = _kan(a, bwT, swT)
    xt3_ref[...] = (jnp.dot(x_ref[...], wa[...], preferred_element_type=jnp.float32)
                    + jnp.dot(x1_ref[...], wb[...], preferred_element_type=jnp.float32)
                    + jnp.dot(x2, wc[...], preferred_element_type=jnp.float32)
                    + ob[...])


_kanb2 = pl.pallas_call(
    _kanb2_body,
    grid=(_N // _BN,),
    in_specs=[
        pl.BlockSpec((2, _BN, _H), lambda i: (0, i, 0)),
        pl.BlockSpec((_H, _H), lambda i: (0, 0)),
        pl.BlockSpec((7, _H, _H), lambda i: (0, 0, 0)),
        pl.BlockSpec((_BN, _H), lambda i: (i, 0)),
        pl.BlockSpec((_BN, _H), lambda i: (i, 0)),
        pl.BlockSpec((_H, _H), lambda i: (0, 0)),
        pl.BlockSpec((_H, _H), lambda i: (0, 0)),
        pl.BlockSpec((_H, _H), lambda i: (0, 0)),
        pl.BlockSpec((1, _H), lambda i: (0, 0)),
    ],
    out_specs=pl.BlockSpec((_BN, _H), lambda i: (i, 0)),
    out_shape=jax.ShapeDtypeStruct((_N, _H), jnp.float32),
)


def _kanc_body(p_ref, bwT, swT, out_ref):
    a = (p_ref[0] + p_ref[1])[:, :_NCLS]
    out_ref[...] = _kan(a, bwT, swT)


_kanc = pl.pallas_call(
    _kanc_body,
    grid=(_N // _BN,),
    in_specs=[
        pl.BlockSpec((2, _BN, _H), lambda i: (0, i, 0)),
        pl.BlockSpec((_NCLS, _NCLS), lambda i: (0, 0)),
        pl.BlockSpec((7, _NCLS, _NCLS), lambda i: (0, 0, 0)),
    ],
    out_specs=pl.BlockSpec((_BN, _NCLS), lambda i: (i, 0)),
    out_shape=jax.ShapeDtypeStruct((_N, _NCLS), jnp.float32),
)


def kernel(x, edge_index, edge_attr,
           l1_eW, l1_eb, l1_nW, l1_nb, l1_bw, l1_sw,
           l2_eW, l2_eb, l2_nW, l2_nb, l2_bw, l2_sw,
           o_eW, o_eb, o_nW, o_nb, o_bw, o_sw,
           grid_h, grid_out):
    npad = _NGP * _GSZ - _E
    ar = jnp.arange(npad, dtype=jnp.int32)
    row1d = jnp.concatenate([edge_index[0], ar % _N])
    col1d = jnp.concatenate([edge_index[1], _N + ar % 112]).reshape(_NGP, _GSZ)

    pad_w = ((0, 0), (0, _H - _NCLS))
    o_eW_p = jnp.pad(o_eW, pad_w)
    o_eb_p = jnp.pad(o_eb, (0, _H - _NCLS))
    o_nW_p = jnp.pad(o_nW, pad_w)
    o_nb_p = jnp.pad(o_nb, (0, _H - _NCLS))

    ea1, ea2, ea3 = _eproj(edge_attr, l1_eW, l2_eW, o_eW_p,
                           l1_eb.reshape(1, _H), l2_eb.reshape(1, _H),
                           o_eb_p.reshape(1, _H))

    _sc128 = _sc_message_pass(_H)

    xt1 = _nproj(x, l1_nW, l1_nb.reshape(1, _H))
    p1 = _sc128(xt1, ea1, row1d, col1d)
    x1, xt2 = _kanb1(p1, l1_bw.T, jnp.transpose(l1_sw, (2, 1, 0)),
                     l2_nW, l2_nb.reshape(1, _H))
    p2 = _sc128(xt2, ea2, row1d, col1d)
    xt3 = _kanb2(p2, l2_bw.T, jnp.transpose(l2_sw, (2, 1, 0)), x, x1,
                 o_nW_p[:_H], o_nW_p[_H:2 * _H], o_nW_p[2 * _H:],
                 o_nb_p.reshape(1, _H))
    p3 = _sc128(xt3, ea3, row1d, col1d)
    return _kanc(p3, o_bw.T, jnp.transpose(o_sw, (2, 1, 0)))


# speedup vs baseline: 1.2479x; 1.0745x over previous
import functools

import jax
import jax.numpy as jnp
import numpy as np
from jax import lax
from jax.experimental import pallas as pl
from jax.experimental.pallas import tpu as pltpu
from jax.experimental.pallas import tpu_sc as plsc

_N = 10000
_E = 160000
_DE = 384
_H = 128
_NCLS = 64
_BE = 3200
_BN = 2000
_GSZ = 64
_NGE = _E // _GSZ
_GMAX = 80
_NGP = _GMAX * 32
_NA = 10112


@functools.lru_cache(maxsize=None)
def _sc_message_pass(h):
    mesh = plsc.VectorSubcoreMesh(core_axis_name="c", subcore_axis_name="s")

    @functools.partial(
        pl.kernel,
        out_type=jax.ShapeDtypeStruct((2, _N, h), jnp.float32),
        mesh=mesh,
        scratch_types=[
            pltpu.VMEM((_GMAX * _GSZ,), jnp.int32),
            pltpu.VMEM((_GMAX, _GSZ), jnp.int32),
            pltpu.VMEM((_GSZ, h), jnp.float32),
            pltpu.VMEM((_GSZ, h), jnp.float32),
            pltpu.VMEM((_GSZ, h), jnp.float32),
            pltpu.VMEM((_GSZ, h), jnp.float32),
            pltpu.VMEM_SHARED((_NA, h), jnp.float32),
            pltpu.SemaphoreType.DMA,
            pltpu.SemaphoreType.DMA,
            pltpu.SemaphoreType.DMA,
            pltpu.SemaphoreType.DMA,
        ],
    )
    def mp(xt_hbm, ea_hbm, row_hbm, col_hbm, out_hbm,
           row_v, col_v, ea0, ea1, xr0, xr1, acc,
           gs0, gs1, es0, es1):
        eav = (ea0, ea1)
        xrv = (xr0, xr1)
        gsem = (gs0, gs1)
        esem = (es0, es1)
        c = lax.axis_index("c")
        s = lax.axis_index("s")
        w = c * 16 + s

        def zrow(r, carry):
            for u in range(h // 16):
                xr0[r, pl.ds(u * 16, 16)] = jnp.zeros((16,), jnp.float32)
            return carry
        lax.fori_loop(0, _GSZ, zrow, 0)
        zc_base = s * 10
        zc_cnt = jnp.minimum(10, 158 - zc_base)

        def zchunk(i, carry):
            pltpu.sync_copy(xr0, acc.at[pl.ds((zc_base + i) * _GSZ, _GSZ)])
            return carry
        lax.fori_loop(0, zc_cnt, zchunk, 0)
        plsc.subcore_barrier()

        pltpu.sync_copy(row_hbm.at[pl.ds(w * _GMAX * _GSZ, _GMAX * _GSZ)],
                        row_v)
        pltpu.sync_copy(col_hbm.at[pl.ds(w * _GMAX, _GMAX)], col_v)

        def gather_copy(k, j):
            return pltpu.make_async_copy(
                xt_hbm.at[row_v.at[pl.ds(k * _GSZ, _GSZ)]], xrv[j], gsem[j])

        def ea_copy(k, j):
            base = jnp.minimum((w * _GMAX + k) * _GSZ, _E - _GSZ)
            return pltpu.make_async_copy(
                ea_hbm.at[pl.ds(base, _GSZ)], eav[j], esem[j])

        gather_copy(0, 0).start()
        ea_copy(0, 0).start()
        gather_copy(1, 1).start()
        ea_copy(1, 1).start()

        def body(i, carry):
            for j in (0, 1):
                k = 2 * i + j
                gather_copy(k, j).wait()
                ea_copy(k, j).wait()

                def mul(r, carry2):
                    for u in range(h // 16):
                        sl = pl.ds(16 * u, 16)
                        xrv[j][r, sl] = eav[j][r, sl] * xrv[j][r, sl]
                    return carry2
                lax.fori_loop(0, _GSZ, mul, 0)

                @pl.when(i < _GMAX // 2 - 1)
                def _():
                    ea_copy(k + 2, j).start()
                pltpu.sync_copy(xrv[j], acc.at[col_v.at[k]], add=True)

                @pl.when(i < _GMAX // 2 - 1)
                def _():
                    gather_copy(k + 2, j).start()
            return carry
        lax.fori_loop(0, _GMAX // 2, body, 0)

        plsc.subcore_barrier()
        oc_base = s * 8
        oc_cnt = jnp.minimum(8, 125 - oc_base)

        def ochunk(i, carry):
            pltpu.sync_copy(acc.at[pl.ds((oc_base + i) * 80, 80)],
                            out_hbm.at[c, pl.ds((oc_base + i) * 80, 80)])
            return carry
        lax.fori_loop(0, oc_cnt, ochunk, 0)

    return mp


def _silu(a):
    return a / (1.0 + jnp.exp(-a))


def _spline_bases(a):
    v = 2.0 * a + 5.0
    c = jnp.floor(v)
    t = v - c
    t2 = t * t
    t3 = t2 * t
    w0 = t3 * (1.0 / 6.0)
    w1 = (1.0 + 3.0 * t + 3.0 * t2 - 3.0 * t3) * (1.0 / 6.0)
    w2 = (4.0 - 6.0 * t2 + 3.0 * t3) * (1.0 / 6.0)
    om = 1.0 - t
    w3 = om * om * om * (1.0 / 6.0)
    zero = jnp.zeros_like(a)
    out = []
    for j in range(7):
        b = (jnp.where(c == j, w0, zero) + jnp.where(c == j + 1, w1, zero)
             + jnp.where(c == j + 2, w2, zero) + jnp.where(c == j + 3, w3, zero))
        out.append(b)
    return out


def _kan(a, bwT_ref, swT_ref):
    out = jnp.dot(_silu(a), bwT_ref[...], preferred_element_type=jnp.float32)
    for j, b in enumerate(_spline_bases(a)):
        out = out + jnp.dot(b, swT_ref[j], preferred_element_type=jnp.float32)
    return out


def _eproj_body(ea_ref, w1, w2, w3, b1, b2, b3, o1, o2, o3):
    a = ea_ref[...]
    o1[...] = jnp.dot(a, w1[...], preferred_element_type=jnp.float32) + b1[...]
    o2[...] = jnp.dot(a, w2[...], preferred_element_type=jnp.float32) + b2[...]
    o3[...] = jnp.dot(a, w3[...], preferred_element_type=jnp.float32) + b3[...]


_eproj = pl.pallas_call(
    _eproj_body,
    grid=(_E // _BE,),
    in_specs=[
        pl.BlockSpec((_BE, _DE), lambda i: (i, 0)),
        pl.BlockSpec((_DE, _H), lambda i: (0, 0)),
        pl.BlockSpec((_DE, _H), lambda i: (0, 0)),
        pl.BlockSpec((_DE, _H), lambda i: (0, 0)),
        pl.BlockSpec((1, _H), lambda i: (0, 0)),
        pl.BlockSpec((1, _H), lambda i: (0, 0)),
        pl.BlockSpec((1, _H), lambda i: (0, 0)),
    ],
    out_specs=[
        pl.BlockSpec((_BE, _H), lambda i: (i, 0)),
        pl.BlockSpec((_BE, _H), lambda i: (i, 0)),
        pl.BlockSpec((_BE, _H), lambda i: (i, 0)),
    ],
    out_shape=[
        jax.ShapeDtypeStruct((_E, _H), jnp.float32),
        jax.ShapeDtypeStruct((_E, _H), jnp.float32),
        jax.ShapeDtypeStruct((_E, _H), jnp.float32),
    ],
)


def _nproj_body(x_ref, w_ref, b_ref, o_ref):
    o_ref[...] = (jnp.dot(x_ref[...], w_ref[...],
                          preferred_element_type=jnp.float32) + b_ref[...])


_nproj = pl.pallas_call(
    _nproj_body,
    grid=(_N // _BN,),
    in_specs=[
        pl.BlockSpec((_BN, _H), lambda i: (i, 0)),
        pl.BlockSpec((_H, _H), lambda i: (0, 0)),
        pl.BlockSpec((1, _H), lambda i: (0, 0)),
    ],
    out_specs=pl.BlockSpec((_BN, _H), lambda i: (i, 0)),
    out_shape=jax.ShapeDtypeStruct((_N, _H), jnp.float32),
)


def _kanb1_body(p_ref, bwT, swT, nW, nb, x1_ref, xt2_ref):
    a = p_ref[0] + p_ref[1]
    o = _kan(a, bwT, swT)
    x1_ref[...] = o
    xt2_ref[...] = (jnp.dot(o, nW[...], preferred_element_type=jnp.float32)
                    + nb[...])


_kanb1 = pl.pallas_call(
    _kanb1_body,
    grid=(_N // _BN,),
    in_specs=[
        pl.BlockSpec((2, _BN, _H), lambda i: (0, i, 0)),
        pl.BlockSpec((_H, _H), lambda i: (0, 0)),
        pl.BlockSpec((7, _H, _H), lambda i: (0, 0, 0)),
        pl.BlockSpec((_H, _H), lambda i: (0, 0)),
        pl.BlockSpec((1, _H), lambda i: (0, 0)),
    ],
    out_specs=[
        pl.BlockSpec((_BN, _H), lambda i: (i, 0)),
        pl.BlockSpec((_BN, _H), lambda i: (i, 0)),
    ],
    out_shape=[
        jax.ShapeDtypeStruct((_N, _H), jnp.float32),
        jax.ShapeDtypeStruct((_N, _H), jnp.float32),
    ],
)


def _kanb2_body(p_ref, bwT, swT, x_ref, x1_ref, wa, wb, wc, ob, xt3_ref):
    a = p_ref[0] + p_ref[1]
    x2 = _kan(a, bwT, swT)
    xt3_ref[...] = (jnp.dot(x_ref[...], wa[...], preferred_element_type=jnp.float32)
                    + jnp.dot(x1_ref[...], wb[...], preferred_element_type=jnp.float32)
                    + jnp.dot(x2, wc[...], preferred_element_type=jnp.float32)
                    + ob[...])


_kanb2 = pl.pallas_call(
    _kanb2_body,
    grid=(_N // _BN,),
    in_specs=[
        pl.BlockSpec((2, _BN, _H), lambda i: (0, i, 0)),
        pl.BlockSpec((_H, _H), lambda i: (0, 0)),
        pl.BlockSpec((7, _H, _H), lambda i: (0, 0, 0)),
        pl.BlockSpec((_BN, _H), lambda i: (i, 0)),
        pl.BlockSpec((_BN, _H), lambda i: (i, 0)),
        pl.BlockSpec((_H, _H), lambda i: (0, 0)),
        pl.BlockSpec((_H, _H), lambda i: (0, 0)),
        pl.BlockSpec((_H, _H), lambda i: (0, 0)),
        pl.BlockSpec((1, _H), lambda i: (0, 0)),
    ],
    out_specs=pl.BlockSpec((_BN, _H), lambda i: (i, 0)),
    out_shape=jax.ShapeDtypeStruct((_N, _H), jnp.float32),
)


def _kanc_body(p_ref, bwT, swT, out_ref):
    a = (p_ref[0] + p_ref[1])[:, :_NCLS]
    out_ref[...] = _kan(a, bwT, swT)


_kanc = pl.pallas_call(
    _kanc_body,
    grid=(_N // _BN,),
    in_specs=[
        pl.BlockSpec((2, _BN, _H), lambda i: (0, i, 0)),
        pl.BlockSpec((_NCLS, _NCLS), lambda i: (0, 0)),
        pl.BlockSpec((7, _NCLS, _NCLS), lambda i: (0, 0, 0)),
    ],
    out_specs=pl.BlockSpec((_BN, _NCLS), lambda i: (i, 0)),
    out_shape=jax.ShapeDtypeStruct((_N, _NCLS), jnp.float32),
)


def kernel(x, edge_index, edge_attr,
           l1_eW, l1_eb, l1_nW, l1_nb, l1_bw, l1_sw,
           l2_eW, l2_eb, l2_nW, l2_nb, l2_bw, l2_sw,
           o_eW, o_eb, o_nW, o_nb, o_bw, o_sw,
           grid_h, grid_out):
    npad = _NGP * _GSZ - _E
    ar = jnp.arange(npad, dtype=jnp.int32)
    row1d = jnp.concatenate([edge_index[0], ar % _N])
    col1d = jnp.concatenate([edge_index[1], _N + ar % 112]).reshape(_NGP, _GSZ)

    pad_w = ((0, 0), (0, _H - _NCLS))
    o_eW_p = jnp.pad(o_eW, pad_w)
    o_eb_p = jnp.pad(o_eb, (0, _H - _NCLS))
    o_nW_p = jnp.pad(o_nW, pad_w)
    o_nb_p = jnp.pad(o_nb, (0, _H - _NCLS))

    ea1, ea2, ea3 = _eproj(edge_attr, l1_eW, l2_eW, o_eW_p,
                           l1_eb.reshape(1, _H), l2_eb.reshape(1, _H),
                           o_eb_p.reshape(1, _H))

    _sc128 = _sc_message_pass(_H)

    xt1 = _nproj(x, l1_nW, l1_nb.reshape(1, _H))
    p1 = _sc128(xt1, ea1, row1d, col1d)
    x1, xt2 = _kanb1(p1, l1_bw.T, jnp.transpose(l1_sw, (2, 1, 0)),
                     l2_nW, l2_nb.reshape(1, _H))
    p2 = _sc128(xt2, ea2, row1d, col1d)
    xt3 = _kanb2(p2, l2_bw.T, jnp.transpose(l2_sw, (2, 1, 0)), x, x1,
                 o_nW_p[:_H], o_nW_p[_H:2 * _H], o_nW_p[2 * _H:],
                 o_nb_p.reshape(1, _H))
    p3 = _sc128(xt3, ea3, row1d, col1d)
    return _kanc(p3, o_bw.T, jnp.transpose(o_sw, (2, 1, 0)))


# speedup vs baseline: 1.2536x; 1.0045x over previous
import functools

import jax
import jax.numpy as jnp
import numpy as np
from jax import lax
from jax.experimental import pallas as pl
from jax.experimental.pallas import tpu as pltpu
from jax.experimental.pallas import tpu_sc as plsc

_N = 10000
_E = 160000
_DE = 384
_H = 128
_NCLS = 64
_BE = 6400
_BN = 5000
_GSZ = 64
_NGE = _E // _GSZ
_GMAX = 80
_NGP = _GMAX * 32
_NA = 10112


@functools.lru_cache(maxsize=None)
def _sc_message_pass(h):
    mesh = plsc.VectorSubcoreMesh(core_axis_name="c", subcore_axis_name="s")

    @functools.partial(
        pl.kernel,
        out_type=jax.ShapeDtypeStruct((2, _N, h), jnp.float32),
        mesh=mesh,
        scratch_types=[
            pltpu.VMEM((_GMAX * _GSZ,), jnp.int32),
            pltpu.VMEM((_GMAX, _GSZ), jnp.int32),
            pltpu.VMEM((_GSZ, h), jnp.float32),
            pltpu.VMEM((_GSZ, h), jnp.float32),
            pltpu.VMEM((_GSZ, h), jnp.float32),
            pltpu.VMEM((_GSZ, h), jnp.float32),
            pltpu.VMEM_SHARED((_NA, h), jnp.float32),
            pltpu.SemaphoreType.DMA,
            pltpu.SemaphoreType.DMA,
            pltpu.SemaphoreType.DMA,
            pltpu.SemaphoreType.DMA,
        ],
    )
    def mp(xt_hbm, ea_hbm, row_hbm, col_hbm, out_hbm,
           row_v, col_v, ea0, ea1, xr0, xr1, acc,
           gs0, gs1, es0, es1):
        eav = (ea0, ea1)
        xrv = (xr0, xr1)
        gsem = (gs0, gs1)
        esem = (es0, es1)
        c = lax.axis_index("c")
        s = lax.axis_index("s")
        w = c * 16 + s

        def zrow(r, carry):
            for u in range(h // 16):
                xr0[r, pl.ds(u * 16, 16)] = jnp.zeros((16,), jnp.float32)
            return carry
        lax.fori_loop(0, _GSZ, zrow, 0)
        zc_base = s * 10
        zc_cnt = jnp.minimum(10, 158 - zc_base)

        def zchunk(i, carry):
            pltpu.sync_copy(xr0, acc.at[pl.ds((zc_base + i) * _GSZ, _GSZ)])
            return carry
        lax.fori_loop(0, zc_cnt, zchunk, 0)
        plsc.subcore_barrier()

        pltpu.sync_copy(row_hbm.at[pl.ds(w * _GMAX * _GSZ, _GMAX * _GSZ)],
                        row_v)
        pltpu.sync_copy(col_hbm.at[pl.ds(w * _GMAX, _GMAX)], col_v)

        def gather_copy(k, j):
            return pltpu.make_async_copy(
                xt_hbm.at[row_v.at[pl.ds(k * _GSZ, _GSZ)]], xrv[j], gsem[j])

        def ea_copy(k, j):
            base = jnp.minimum((w * _GMAX + k) * _GSZ, _E - _GSZ)
            return pltpu.make_async_copy(
                ea_hbm.at[pl.ds(base, _GSZ)], eav[j], esem[j])

        gather_copy(0, 0).start()
        ea_copy(0, 0).start()
        gather_copy(1, 1).start()
        ea_copy(1, 1).start()

        def body(i, carry):
            for j in (0, 1):
                k = 2 * i + j
                gather_copy(k, j).wait()
                ea_copy(k, j).wait()

                def mul(r, carry2):
                    for u in range(h // 16):
                        sl = pl.ds(16 * u, 16)
                        xrv[j][r, sl] = eav[j][r, sl] * xrv[j][r, sl]
                    return carry2
                lax.fori_loop(0, _GSZ, mul, 0)

                @pl.when(i < _GMAX // 2 - 1)
                def _():
                    ea_copy(k + 2, j).start()
                pltpu.sync_copy(xrv[j], acc.at[col_v.at[k]], add=True)

                @pl.when(i < _GMAX // 2 - 1)
                def _():
                    gather_copy(k + 2, j).start()
            return carry
        lax.fori_loop(0, _GMAX // 2, body, 0)

        plsc.subcore_barrier()
        oc_base = s * 8
        oc_cnt = jnp.minimum(8, 125 - oc_base)

        def ochunk(i, carry):
            pltpu.sync_copy(acc.at[pl.ds((oc_base + i) * 80, 80)],
                            out_hbm.at[c, pl.ds((oc_base + i) * 80, 80)])
            return carry
        lax.fori_loop(0, oc_cnt, ochunk, 0)

    return mp


def _silu(a):
    return a / (1.0 + jnp.exp(-a))


def _spline_bases(a):
    v = 2.0 * a + 5.0
    c = jnp.floor(v)
    t = v - c
    t2 = t * t
    t3 = t2 * t
    w0 = t3 * (1.0 / 6.0)
    w1 = (1.0 + 3.0 * t + 3.0 * t2 - 3.0 * t3) * (1.0 / 6.0)
    w2 = (4.0 - 6.0 * t2 + 3.0 * t3) * (1.0 / 6.0)
    om = 1.0 - t
    w3 = om * om * om * (1.0 / 6.0)
    zero = jnp.zeros_like(a)
    out = []
    for j in range(7):
        b = (jnp.where(c == j, w0, zero) + jnp.where(c == j + 1, w1, zero)
             + jnp.where(c == j + 2, w2, zero) + jnp.where(c == j + 3, w3, zero))
        out.append(b)
    return out


def _kan(a, bwT_ref, swT_ref):
    out = jnp.dot(_silu(a), bwT_ref[...], preferred_element_type=jnp.float32)
    for j, b in enumerate(_spline_bases(a)):
        out = out + jnp.dot(b, swT_ref[j], preferred_element_type=jnp.float32)
    return out


def _eproj_body(ea_ref, w1, w2, w3, b1, b2, b3, o1, o2, o3):
    a = ea_ref[...]
    o1[...] = jnp.dot(a, w1[...], preferred_element_type=jnp.float32) + b1[...]
    o2[...] = jnp.dot(a, w2[...], preferred_element_type=jnp.float32) + b2[...]
    o3[...] = jnp.dot(a, w3[...], preferred_element_type=jnp.float32) + b3[...]


_eproj = pl.pallas_call(
    _eproj_body,
    grid=(_E // _BE,),
    in_specs=[
        pl.BlockSpec((_BE, _DE), lambda i: (i, 0)),
        pl.BlockSpec((_DE, _H), lambda i: (0, 0)),
        pl.BlockSpec((_DE, _H), lambda i: (0, 0)),
        pl.BlockSpec((_DE, _H), lambda i: (0, 0)),
        pl.BlockSpec((1, _H), lambda i: (0, 0)),
        pl.BlockSpec((1, _H), lambda i: (0, 0)),
        pl.BlockSpec((1, _H), lambda i: (0, 0)),
    ],
    out_specs=[
        pl.BlockSpec((_BE, _H), lambda i: (i, 0)),
        pl.BlockSpec((_BE, _H), lambda i: (i, 0)),
        pl.BlockSpec((_BE, _H), lambda i: (i, 0)),
    ],
    out_shape=[
        jax.ShapeDtypeStruct((_E, _H), jnp.float32),
        jax.ShapeDtypeStruct((_E, _H), jnp.float32),
        jax.ShapeDtypeStruct((_E, _H), jnp.float32),
    ],
)


def _nproj_body(x_ref, w_ref, b_ref, o_ref):
    o_ref[...] = (jnp.dot(x_ref[...], w_ref[...],
                          preferred_element_type=jnp.float32) + b_ref[...])


_nproj = pl.pallas_call(
    _nproj_body,
    grid=(_N // _BN,),
    in_specs=[
        pl.BlockSpec((_BN, _H), lambda i: (i, 0)),
        pl.BlockSpec((_H, _H), lambda i: (0, 0)),
        pl.BlockSpec((1, _H), lambda i: (0, 0)),
    ],
    out_specs=pl.BlockSpec((_BN, _H), lambda i: (i, 0)),
    out_shape=jax.ShapeDtypeStruct((_N, _H), jnp.float32),
)


def _kanb1_body(p_ref, bwT, swT, nW, nb, x1_ref, xt2_ref):
    a = p_ref[0] + p_ref[1]
    o = _kan(a, bwT, swT)
    x1_ref[...] = o
    xt2_ref[...] = (jnp.dot(o, nW[...], preferred_element_type=jnp.float32)
                    + nb[...])


_kanb1 = pl.pallas_call(
    _kanb1_body,
    grid=(_N // _BN,),
    in_specs=[
        pl.BlockSpec((2, _BN, _H), lambda i: (0, i, 0)),
        pl.BlockSpec((_H, _H), lambda i: (0, 0)),
        pl.BlockSpec((7, _H, _H), lambda i: (0, 0, 0)),
        pl.BlockSpec((_H, _H), lambda i: (0, 0)),
        pl.BlockSpec((1, _H), lambda i: (0, 0)),
    ],
    out_specs=[
        pl.BlockSpec((_BN, _H), lambda i: (i, 0)),
        pl.BlockSpec((_BN, _H), lambda i: (i, 0)),
    ],
    out_shape=[
        jax.ShapeDtypeStruct((_N, _H), jnp.float32),
        jax.ShapeDtypeStruct((_N, _H), jnp.float32),
    ],
)


def _kanb2_body(p_ref, bwT, swT, x_ref, x1_ref, wa, wb, wc, ob, xt3_ref):
    a = p_ref[0] + p_ref[1]
    x2 = _kan(a, bwT, swT)
    xt3_ref[...] = (jnp.dot(x_ref[...], wa[...], preferred_element_type=jnp.float32)
                    + jnp.dot(x1_ref[...], wb[...], preferred_element_type=jnp.float32)
                    + jnp.dot(x2, wc[...], preferred_element_type=jnp.float32)
                    + ob[...])


_kanb2 = pl.pallas_call(
    _kanb2_body,
    grid=(_N // _BN,),
    in_specs=[
        pl.BlockSpec((2, _BN, _H), lambda i: (0, i, 0)),
        pl.BlockSpec((_H, _H), lambda i: (0, 0)),
        pl.BlockSpec((7, _H, _H), lambda i: (0, 0, 0)),
        pl.BlockSpec((_BN, _H), lambda i: (i, 0)),
        pl.BlockSpec((_BN, _H), lambda i: (i, 0)),
        pl.BlockSpec((_H, _H), lambda i: (0, 0)),
        pl.BlockSpec((_H, _H), lambda i: (0, 0)),
        pl.BlockSpec((_H, _H), lambda i: (0, 0)),
        pl.BlockSpec((1, _H), lambda i: (0, 0)),
    ],
    out_specs=pl.BlockSpec((_BN, _H), lambda i: (i, 0)),
    out_shape=jax.ShapeDtypeStruct((_N, _H), jnp.float32),
)


def _kanc_body(p_ref, bwT, swT, out_ref):
    a = (p_ref[0] + p_ref[1])[:, :_NCLS]
    out_ref[...] = _kan(a, bwT, swT)


_kanc = pl.pallas_call(
    _kanc_body,
    grid=(_N // _BN,),
    in_specs=[
        pl.BlockSpec((2, _BN, _H), lambda i: (0, i, 0)),
        pl.BlockSpec((_NCLS, _NCLS), lambda i: (0, 0)),
        pl.BlockSpec((7, _NCLS, _NCLS), lambda i: (0, 0, 0)),
    ],
    out_specs=pl.BlockSpec((_BN, _NCLS), lambda i: (i, 0)),
    out_shape=jax.ShapeDtypeStruct((_N, _NCLS), jnp.float32),
)


def kernel(x, edge_index, edge_attr,
           l1_eW, l1_eb, l1_nW, l1_nb, l1_bw, l1_sw,
           l2_eW, l2_eb, l2_nW, l2_nb, l2_bw, l2_sw,
           o_eW, o_eb, o_nW, o_nb, o_bw, o_sw,
           grid_h, grid_out):
    npad = _NGP * _GSZ - _E
    ar = jnp.arange(npad, dtype=jnp.int32)
    row1d = jnp.concatenate([edge_index[0], ar % _N])
    col1d = jnp.concatenate([edge_index[1], _N + ar % 112]).reshape(_NGP, _GSZ)

    pad_w = ((0, 0), (0, _H - _NCLS))
    o_eW_p = jnp.pad(o_eW, pad_w)
    o_eb_p = jnp.pad(o_eb, (0, _H - _NCLS))
    o_nW_p = jnp.pad(o_nW, pad_w)
    o_nb_p = jnp.pad(o_nb, (0, _H - _NCLS))

    ea1, ea2, ea3 = _eproj(edge_attr, l1_eW, l2_eW, o_eW_p,
                           l1_eb.reshape(1, _H), l2_eb.reshape(1, _H),
                           o_eb_p.reshape(1, _H))

    _sc128 = _sc_message_pass(_H)

    xt1 = _nproj(x, l1_nW, l1_nb.reshape(1, _H))
    p1 = _sc128(xt1, ea1, row1d, col1d)
    x1, xt2 = _kanb1(p1, l1_bw.T, jnp.transpose(l1_sw, (2, 1, 0)),
                     l2_nW, l2_nb.reshape(1, _H))
    p2 = _sc128(xt2, ea2, row1d, col1d)
    xt3 = _kanb2(p2, l2_bw.T, jnp.transpose(l2_sw, (2, 1, 0)), x, x1,
                 o_nW_p[:_H], o_nW_p[_H:2 * _H], o_nW_p[2 * _H:],
                 o_nb_p.reshape(1, _H))
    p3 = _sc128(xt3, ea3, row1d, col1d)
    return _kanc(p3, o_bw.T, jnp.transpose(o_sw, (2, 1, 0)))
